# Initial kernel scaffold; baseline (speedup 1.0000x reference)
import jax
import jax.numpy as jnp
from jax.experimental import pallas as pl


def kernel(x_vals, x_cons, x_obj, pe_vals, pe_cons, pe_obj, src_c2v, dst_c2v, src_v2c, dst_v2c, params):
    raise NotImplementedError("write your pallas kernel here")


# speedup vs baseline: 1.5809x; 1.5809x over previous
import functools

import jax
import jax.numpy as jnp
from jax.experimental import pallas as pl

EPS = 1e-7
HID = 128


def _mlp_body(g_ref, w1_ref, b1_ref, w2_ref, b2_ref, o_ref):
    g = g_ref[...]
    h = jnp.maximum(
        jax.lax.dot_general(g, w1_ref[...], (((1,), (0,)), ((), ())),
                            preferred_element_type=jnp.float32) + b1_ref[...], 0.0)
    o_ref[...] = jax.lax.dot_general(h, w2_ref[...], (((1,), (0,)), ((), ())),
                                     preferred_element_type=jnp.float32) + b2_ref[...]


@functools.partial(jax.jit, static_argnames=("blk",))
def _mlp(g, W1, b1, W2, b2, blk=2000):
    n, k = g.shape
    k2 = W1.shape[1]
    ko = W2.shape[1]
    grid = (n // blk,)
    return pl.pallas_call(
        _mlp_body,
        grid=grid,
        in_specs=[
            pl.BlockSpec((blk, k), lambda i: (i, 0)),
            pl.BlockSpec((k, k2), lambda i: (0, 0)),
            pl.BlockSpec((1, k2), lambda i: (0, 0)),
            pl.BlockSpec((k2, ko), lambda i: (0, 0)),
            pl.BlockSpec((1, ko), lambda i: (0, 0)),
        ],
        out_specs=pl.BlockSpec((blk, ko), lambda i: (i, 0)),
        out_shape=jax.ShapeDtypeStruct((n, ko), jnp.float32),
    )(g, W1, b1.reshape(1, -1), W2, b2.reshape(1, -1))


def kernel(x_vals, x_cons, x_obj, pe_vals, pe_cons, pe_obj, src_c2v, dst_c2v, src_v2c, dst_v2c, params):
    p = params

    def enc(x, bp, We, be):
        a = x @ We + be
        b = jnp.broadcast_to(bp, (x.shape[0], bp.shape[0]))
        return jnp.maximum(jnp.concatenate([a, b], axis=1), 0.0)

    hv = enc(x_vals, p["b_pe_vals"], p["W_enc_vals"], p["b_enc_vals"])
    hc = enc(x_cons, p["b_pe_cons"], p["W_enc_cons"], p["b_enc_cons"])

    def conv(x_src, x_dst, src, dst, W1, b1, W2, b2):
        y = jnp.maximum(x_src, 0.0) + EPS
        M = jnp.max(y, axis=0, keepdims=True)
        P = jnp.exp(y - M)
        Q = P * y
        den = jax.ops.segment_sum(P[src], dst, num_segments=x_dst.shape[0])
        num = jax.ops.segment_sum(Q[src], dst, num_segments=x_dst.shape[0])
        g = num / (den + 1e-16) + x_dst
        return _mlp(g, W1, b1, W2, b2)

    hid_c, hid_v = [], []
    for _ in range(2):
        h2_v = conv(hc, hv, src_c2v, dst_c2v, p["W1_c2v"], p["b1_c2v"], p["W2_c2v"], p["b2_c2v"])
        h2_c = conv(hv, hc, src_v2c, dst_v2c, p["W1_v2c"], p["b1_v2c"], p["W2_v2c"], p["b2_v2c"])
        hid_c.append(h2_c)
        hid_v.append(h2_v)
        hv = (jnp.maximum(h2_v, 0.0) + hv) / 2.0
        hc = (jnp.maximum(h2_c, 0.0) + hc) / 2.0

    vals = jnp.concatenate(hid_v, axis=0)
    cons = jnp.concatenate(hid_c, axis=0)
    vals = _mlp(vals, p["Wpv1"], p["bpv1"], p["Wpv2"], p["bpv2"])
    cons = _mlp(cons, p["Wpc1"], p["bpc1"], p["Wpc2"], p["bpc2"])
    nv = x_vals.shape[0]
    nc = x_cons.shape[0]
    out_vals = jnp.stack([vals[:nv], vals[nv:]], axis=1)
    out_cons = jnp.stack([cons[:nc], cons[nc:]], axis=1)[..., 0]
    return (out_vals, out_cons)


# speedup vs baseline: 12.4057x; 7.8474x over previous
import functools

import jax
import jax.numpy as jnp
from jax import lax
from jax.experimental import pallas as pl
from jax.experimental.pallas import tpu as pltpu
from jax.experimental.pallas import tpu_sc as plsc

EPS = 1e-7
HID = 128

NTILES = 16
K = 128
EPT = 37888
EP = NTILES * EPT
ZROWS = 8


def _sc_aggregate(nsrc, nranges, rng, seg, nbuf=2):
    ar = rng + 2 * K
    nseg = EPT // seg
    nbmax = (seg + K + K - 1) // K
    zpt = ar // NTILES
    opt = rng // NTILES
    ngrp = (nbmax + nbuf - 1) // nbuf

    def body(t_hbm, src_hbm, dst_hbm, out_hbm, accum, segsrc, segdst, cg2, cd2, zbuf, *rest):
        rbufs = rest[:nbuf]
        sems = rest[nbuf:]
        c = lax.axis_index("c")
        s = lax.axis_index("s")
        bias = c * nsrc
        lanes = lax.iota(jnp.int32, 16)

        def zrow(i, _):
            def zcol(j, _):
                zbuf[i, pl.ds(j * 16, 16)] = jnp.zeros((16,), jnp.float32)
                return 0
            return lax.fori_loop(0, HID // 16, zcol, 0)
        lax.fori_loop(0, ZROWS, zrow, 0)

        for r in range(nranges):
            lo = r * rng
            def zacc(z, _):
                pltpu.sync_copy(zbuf, accum.at[pl.ds(s * zpt + z * ZROWS, ZROWS)])
                return 0
            lax.fori_loop(0, zpt // ZROWS, zacc, 0)
            plsc.subcore_barrier()

            def seg_body(sg, _):
                base = s * EPT + sg * seg
                pltpu.sync_copy(src_hbm.at[pl.ds(base, seg)], segsrc)
                pltpu.sync_copy(dst_hbm.at[pl.ds(base, seg)], segdst)

                def compact(i, n16):
                    src16 = segsrc[pl.ds(i * 16, 16)]
                    dst16 = segdst[pl.ds(i * 16, 16)]
                    msk = (dst16 >= lo) & (dst16 < lo + rng)
                    mi = msk.astype(jnp.int32)
                    pos = n16 + plsc.cumsum(mi) - mi
                    row = lax.shift_right_logical(pos, 7)
                    col = pos & (K - 1)
                    plsc.store_scatter(cg2, [row, col], src16 + bias, mask=msk)
                    plsc.store_scatter(cd2, [row, col], dst16 - lo, mask=msk)
                    return n16 + plsc.all_reduce_population_count(msk)
                n16 = lax.fori_loop(0, seg // 16, compact,
                                    jnp.zeros((16,), jnp.int32))
                for t in range(K // 16):
                    pos = n16 + lanes + t * 16
                    row = lax.shift_right_logical(pos, 7)
                    col = pos & (K - 1)
                    plsc.store_scatter(cg2, [row, col], bias + col)
                    plsc.store_scatter(cd2, [row, col], rng + col)
                n_sc = jnp.max(n16)
                nb = lax.shift_right_logical(n_sc + (K - 1), 7)

                for j in range(nbuf):
                    @pl.when(j < nb)
                    def _():
                        pltpu.async_copy(t_hbm.at[cg2.at[j]], rbufs[j], sems[j])

                def grp(g, _):
                    for j in range(nbuf):
                        b = g * nbuf + j

                        @pl.when(b < nb)
                        def _():
                            pltpu.make_async_copy(
                                t_hbm.at[cg2.at[b]], rbufs[j], sems[j]).wait()
                            pltpu.sync_copy(rbufs[j], accum.at[cd2.at[b]], add=True)

                            @pl.when(b + nbuf < nb)
                            def _():
                                pltpu.async_copy(
                                    t_hbm.at[cg2.at[b + nbuf]], rbufs[j], sems[j])
                    return 0
                lax.fori_loop(0, ngrp, grp, 0)
                return 0
            lax.fori_loop(0, nseg, seg_body, 0)
            plsc.subcore_barrier()

            pltpu.sync_copy(accum.at[pl.ds(s * opt, opt)],
                            out_hbm.at[pl.ds((c * nranges + r) * rng + s * opt, opt)])
            plsc.subcore_barrier()

    mesh = plsc.VectorSubcoreMesh(core_axis_name="c", subcore_axis_name="s")
    return pl.kernel(
        body,
        out_type=jax.ShapeDtypeStruct((2 * nranges * rng, HID), jnp.float32),
        mesh=mesh,
        compiler_params=pltpu.CompilerParams(needs_layout_passes=False),
        scratch_types=(
            [pltpu.VMEM_SHARED((ar, HID), jnp.float32),
             pltpu.VMEM((seg,), jnp.int32),
             pltpu.VMEM((seg,), jnp.int32),
             pltpu.VMEM((nbmax + 2, K), jnp.int32),
             pltpu.VMEM((nbmax + 2, K), jnp.int32),
             pltpu.VMEM((ZROWS, HID), jnp.float32)]
            + [pltpu.VMEM((K, HID), jnp.float32) for _ in range(nbuf)]
            + [pltpu.SemaphoreType.DMA for _ in range(nbuf)]
        ),
    )


def _pad_edges(src, dst, ndst):
    npad = EP - src.shape[0]
    fill = jnp.arange(npad, dtype=jnp.int32) % K
    gsrc = jnp.concatenate([src.astype(jnp.int32), fill])
    dpad = jnp.concatenate([dst.astype(jnp.int32), ndst + fill])
    return gsrc, dpad


def _mlp_body(g_ref, w1_ref, b1_ref, w2_ref, b2_ref, o_ref):
    g = g_ref[...]
    h = jnp.maximum(
        jax.lax.dot_general(g, w1_ref[...], (((1,), (0,)), ((), ())),
                            preferred_element_type=jnp.float32) + b1_ref[...], 0.0)
    o_ref[...] = jax.lax.dot_general(h, w2_ref[...], (((1,), (0,)), ((), ())),
                                     preferred_element_type=jnp.float32) + b2_ref[...]


@functools.partial(jax.jit, static_argnames=("blk",))
def _mlp(g, W1, b1, W2, b2, blk=2000):
    n, k = g.shape
    k2 = W1.shape[1]
    ko = W2.shape[1]
    grid = (n // blk,)
    return pl.pallas_call(
        _mlp_body,
        grid=grid,
        in_specs=[
            pl.BlockSpec((blk, k), lambda i: (i, 0)),
            pl.BlockSpec((k, k2), lambda i: (0, 0)),
            pl.BlockSpec((1, k2), lambda i: (0, 0)),
            pl.BlockSpec((k2, ko), lambda i: (0, 0)),
            pl.BlockSpec((1, ko), lambda i: (0, 0)),
        ],
        out_specs=pl.BlockSpec((blk, ko), lambda i: (i, 0)),
        out_shape=jax.ShapeDtypeStruct((n, ko), jnp.float32),
    )(g, W1, b1.reshape(1, -1), W2, b2.reshape(1, -1))


def kernel(x_vals, x_cons, x_obj, pe_vals, pe_cons, pe_obj, src_c2v, dst_c2v, src_v2c, dst_v2c, params):
    p = params

    def enc(x, bp, We, be):
        a = x @ We + be
        b = jnp.broadcast_to(bp, (x.shape[0], bp.shape[0]))
        return jnp.maximum(jnp.concatenate([a, b], axis=1), 0.0)

    hv = enc(x_vals, p["b_pe_vals"], p["W_enc_vals"], p["b_enc_vals"])
    hc = enc(x_cons, p["b_pe_cons"], p["W_enc_cons"], p["b_enc_cons"])

    nv = x_vals.shape[0]
    nc = x_cons.shape[0]
    agg_c2v = _sc_aggregate(nsrc=nc, nranges=5, rng=8192, seg=4736)
    agg_v2c = _sc_aggregate(nsrc=nv, nranges=1, rng=10240, seg=2368)
    g_c2v, d_c2v = _pad_edges(src_c2v, dst_c2v, nv)
    g_v2c, d_v2c = _pad_edges(src_v2c, dst_v2c, nc)

    def conv(x_src, x_dst, gsrc, dpad, agg, nranges, W1, b1, W2, b2):
        ndst = x_dst.shape[0]
        y = jnp.maximum(x_src, 0.0) + EPS
        M = jnp.max(y, axis=0, keepdims=True)
        P = jnp.exp(y - M)
        Q = P * y
        T = jnp.concatenate([P, Q], axis=0)
        res = agg(T, gsrc, dpad)
        res = res.reshape(2, -1, HID)[:, :ndst]
        den, num = res[0], res[1]
        g = num / (den + 1e-16) + x_dst
        return _mlp(g, W1, b1, W2, b2)

    hid_c, hid_v = [], []
    for _ in range(2):
        h2_v = conv(hc, hv, g_c2v, d_c2v, agg_c2v, 5, p["W1_c2v"], p["b1_c2v"], p["W2_c2v"], p["b2_c2v"])
        h2_c = conv(hv, hc, g_v2c, d_v2c, agg_v2c, 1, p["W1_v2c"], p["b1_v2c"], p["W2_v2c"], p["b2_v2c"])
        hid_c.append(h2_c)
        hid_v.append(h2_v)
        hv = (jnp.maximum(h2_v, 0.0) + hv) / 2.0
        hc = (jnp.maximum(h2_c, 0.0) + hc) / 2.0

    vals = jnp.concatenate(hid_v, axis=0)
    cons = jnp.concatenate(hid_c, axis=0)
    vals = _mlp(vals, p["Wpv1"], p["bpv1"], p["Wpv2"], p["bpv2"])
    cons = _mlp(cons, p["Wpc1"], p["bpc1"], p["Wpc2"], p["bpc2"])
    out_vals = jnp.stack([vals[:nv], vals[nv:]], axis=1)
    out_cons = jnp.stack([cons[:nc], cons[nc:]], axis=1)[..., 0]
    return (out_vals, out_cons)


# speedup vs baseline: 13.1025x; 1.0562x over previous
import functools

import jax
import jax.numpy as jnp
from jax import lax
from jax.experimental import pallas as pl
from jax.experimental.pallas import tpu as pltpu
from jax.experimental.pallas import tpu_sc as plsc

EPS = 1e-7
HID = 128

NTILES = 16
K = 128
EPT = 37888
EP = NTILES * EPT
ZROWS = 8


def _sc_aggregate(nsrc, nranges, rng, seg, nbuf=2):
    ar = rng + 2 * K
    nseg = EPT // seg
    nbmax = (seg + K + K - 1) // K
    zpt = ar // NTILES
    opt = rng // NTILES
    ngrp = (nbmax + nbuf - 1) // nbuf

    def body(t_hbm, src_hbm, dst_hbm, out_hbm, accum, segsrc, segdst, cg2, cd2, zbuf, *rest):
        rbufs = rest[:nbuf]
        sems = rest[nbuf:]
        c = lax.axis_index("c")
        s = lax.axis_index("s")
        bias = c * nsrc
        lanes = lax.iota(jnp.int32, 16)

        def zrow(i, _):
            def zcol(j, _):
                zbuf[i, pl.ds(j * 16, 16)] = jnp.zeros((16,), jnp.float32)
                return 0
            return lax.fori_loop(0, HID // 16, zcol, 0)
        lax.fori_loop(0, ZROWS, zrow, 0)

        for r in range(nranges):
            lo = r * rng
            def zacc(z, _):
                pltpu.sync_copy(zbuf, accum.at[pl.ds(s * zpt + z * ZROWS, ZROWS)])
                return 0
            lax.fori_loop(0, zpt // ZROWS, zacc, 0)
            plsc.subcore_barrier()

            def seg_body(sg, _):
                base = s * EPT + sg * seg
                pltpu.sync_copy(src_hbm.at[pl.ds(base, seg)], segsrc)
                pltpu.sync_copy(dst_hbm.at[pl.ds(base, seg)], segdst)

                def compact(i, n16):
                    src16 = segsrc[pl.ds(i * 16, 16)]
                    dst16 = segdst[pl.ds(i * 16, 16)]
                    msk = (dst16 >= lo) & (dst16 < lo + rng)
                    mi = msk.astype(jnp.int32)
                    pos = n16 + plsc.cumsum(mi) - mi
                    row = lax.shift_right_logical(pos, 7)
                    col = pos & (K - 1)
                    plsc.store_scatter(cg2, [row, col], src16 + bias, mask=msk)
                    plsc.store_scatter(cd2, [row, col], dst16 - lo, mask=msk)
                    return n16 + plsc.all_reduce_population_count(msk)
                n16 = lax.fori_loop(0, seg // 16, compact,
                                    jnp.zeros((16,), jnp.int32))
                for t in range(K // 16):
                    pos = n16 + lanes + t * 16
                    row = lax.shift_right_logical(pos, 7)
                    col = pos & (K - 1)
                    plsc.store_scatter(cg2, [row, col], bias + col)
                    plsc.store_scatter(cd2, [row, col], rng + col)
                n_sc = jnp.max(n16)
                nb = lax.shift_right_logical(n_sc + (K - 1), 7)

                for j in range(nbuf):
                    @pl.when(j < nb)
                    def _():
                        pltpu.async_copy(t_hbm.at[cg2.at[j]], rbufs[j], sems[j])

                def grp(g, _):
                    for j in range(nbuf):
                        b = g * nbuf + j

                        @pl.when(b < nb)
                        def _():
                            pltpu.make_async_copy(
                                t_hbm.at[cg2.at[b]], rbufs[j], sems[j]).wait()
                            pltpu.sync_copy(rbufs[j], accum.at[cd2.at[b]], add=True)

                            @pl.when(b + nbuf < nb)
                            def _():
                                pltpu.async_copy(
                                    t_hbm.at[cg2.at[b + nbuf]], rbufs[j], sems[j])
                    return 0
                lax.fori_loop(0, ngrp, grp, 0)
                return 0
            lax.fori_loop(0, nseg, seg_body, 0)
            plsc.subcore_barrier()

            pltpu.sync_copy(accum.at[pl.ds(s * opt, opt)],
                            out_hbm.at[pl.ds((c * nranges + r) * rng + s * opt, opt)])
            plsc.subcore_barrier()

    mesh = plsc.VectorSubcoreMesh(core_axis_name="c", subcore_axis_name="s")
    return pl.kernel(
        body,
        out_type=jax.ShapeDtypeStruct((2 * nranges * rng, HID), jnp.float32),
        mesh=mesh,
        compiler_params=pltpu.CompilerParams(needs_layout_passes=False),
        scratch_types=(
            [pltpu.VMEM_SHARED((ar, HID), jnp.float32),
             pltpu.VMEM((seg,), jnp.int32),
             pltpu.VMEM((seg,), jnp.int32),
             pltpu.VMEM((nbmax + 2, K), jnp.int32),
             pltpu.VMEM((nbmax + 2, K), jnp.int32),
             pltpu.VMEM((ZROWS, HID), jnp.float32)]
            + [pltpu.VMEM((K, HID), jnp.float32) for _ in range(nbuf)]
            + [pltpu.SemaphoreType.DMA for _ in range(nbuf)]
        ),
    )


def _pad_edges(src, dst, ndst):
    npad = EP - src.shape[0]
    fill = jnp.arange(npad, dtype=jnp.int32) % K
    gsrc = jnp.concatenate([src.astype(jnp.int32), fill])
    dpad = jnp.concatenate([dst.astype(jnp.int32), ndst + fill])
    return gsrc, dpad


def _mlp_body(g_ref, w1_ref, b1_ref, w2_ref, b2_ref, o_ref):
    g = g_ref[...]
    h = jnp.maximum(
        jax.lax.dot_general(g, w1_ref[...], (((1,), (0,)), ((), ())),
                            preferred_element_type=jnp.float32) + b1_ref[...], 0.0)
    o_ref[...] = jax.lax.dot_general(h, w2_ref[...], (((1,), (0,)), ((), ())),
                                     preferred_element_type=jnp.float32) + b2_ref[...]


BLK = 1024


def _mlp(g, W1, b1, W2, b2):
    n, k = g.shape
    k2 = W1.shape[1]
    ko = W2.shape[1]
    return pl.pallas_call(
        _mlp_body,
        grid=(n // BLK,),
        in_specs=[
            pl.BlockSpec((BLK, k), lambda i: (i, 0)),
            pl.BlockSpec((k, k2), lambda i: (0, 0)),
            pl.BlockSpec((1, k2), lambda i: (0, 0)),
            pl.BlockSpec((k2, ko), lambda i: (0, 0)),
            pl.BlockSpec((1, ko), lambda i: (0, 0)),
        ],
        out_specs=pl.BlockSpec((BLK, ko), lambda i: (i, 0)),
        out_shape=jax.ShapeDtypeStruct((n, ko), jnp.float32),
    )(g, W1, b1.reshape(1, -1), W2, b2.reshape(1, -1))


def _enc(xp, We, be, bp):
    n = xp.shape[0]
    fin = xp.shape[1]
    h2 = We.shape[1]

    def body(x_ref, we_ref, be_ref, bp_ref, o_ref):
        a = jax.lax.dot_general(x_ref[...], we_ref[...], (((1,), (0,)), ((), ())),
                                preferred_element_type=jnp.float32) + be_ref[...]
        b = jnp.broadcast_to(bp_ref[...], (BLK, h2))
        o_ref[...] = jnp.maximum(jnp.concatenate([a, b], axis=1), 0.0)

    return pl.pallas_call(
        body,
        grid=(n // BLK,),
        in_specs=[
            pl.BlockSpec((BLK, fin), lambda i: (i, 0)),
            pl.BlockSpec((fin, h2), lambda i: (0, 0)),
            pl.BlockSpec((1, h2), lambda i: (0, 0)),
            pl.BlockSpec((1, h2), lambda i: (0, 0)),
        ],
        out_specs=pl.BlockSpec((BLK, 2 * h2), lambda i: (i, 0)),
        out_shape=jax.ShapeDtypeStruct((n, 2 * h2), jnp.float32),
    )(xp, We, be.reshape(1, -1), bp.reshape(1, -1))


def _prep(x, M):
    n = x.shape[0]
    nblk = n // BLK

    def body(x_ref, m_ref, t_ref):
        i = pl.program_id(0)
        y = jnp.maximum(x_ref[...], 0.0) + EPS
        p = jnp.exp(y - m_ref[...])
        t_ref[...] = jnp.where(i >= nblk, p * y, p)

    return pl.pallas_call(
        body,
        grid=(2 * nblk,),
        in_specs=[
            pl.BlockSpec((BLK, HID), lambda i: (jax.lax.rem(i, nblk), 0)),
            pl.BlockSpec((1, HID), lambda i: (0, 0)),
        ],
        out_specs=pl.BlockSpec((BLK, HID), lambda i: (i, 0)),
        out_shape=jax.ShapeDtypeStruct((2 * n, HID), jnp.float32),
    )(x, M)


def _post(res, x_dst, W1, b1, W2, b2):
    n = x_dst.shape[0]
    nblk = n // BLK

    def body(den_ref, num_ref, x_ref, w1_ref, b1_ref, w2_ref, b2_ref,
             h2_ref, xn_ref):
        x = x_ref[...]
        g = num_ref[...] / (den_ref[...] + 1e-16) + x
        h = jnp.maximum(
            jax.lax.dot_general(g, w1_ref[...], (((1,), (0,)), ((), ())),
                                preferred_element_type=jnp.float32) + b1_ref[...], 0.0)
        h2 = jax.lax.dot_general(h, w2_ref[...], (((1,), (0,)), ((), ())),
                                 preferred_element_type=jnp.float32) + b2_ref[...]
        h2_ref[...] = h2
        xn_ref[...] = (jnp.maximum(h2, 0.0) + x) * 0.5

    return pl.pallas_call(
        body,
        grid=(nblk,),
        in_specs=[
            pl.BlockSpec((BLK, HID), lambda i: (i, 0)),
            pl.BlockSpec((BLK, HID), lambda i: (i + nblk, 0)),
            pl.BlockSpec((BLK, HID), lambda i: (i, 0)),
            pl.BlockSpec((HID, 2 * HID), lambda i: (0, 0)),
            pl.BlockSpec((1, 2 * HID), lambda i: (0, 0)),
            pl.BlockSpec((2 * HID, HID), lambda i: (0, 0)),
            pl.BlockSpec((1, HID), lambda i: (0, 0)),
        ],
        out_specs=[pl.BlockSpec((BLK, HID), lambda i: (i, 0)),
                   pl.BlockSpec((BLK, HID), lambda i: (i, 0))],
        out_shape=[jax.ShapeDtypeStruct((n, HID), jnp.float32),
                   jax.ShapeDtypeStruct((n, HID), jnp.float32)],
    )(res, res, x_dst, W1, b1.reshape(1, -1), W2, b2.reshape(1, -1))


LV = 40960
LC = 10240


def kernel(x_vals, x_cons, x_obj, pe_vals, pe_cons, pe_obj, src_c2v, dst_c2v, src_v2c, dst_v2c, params):
    p = params
    nv = x_vals.shape[0]
    nc = x_cons.shape[0]

    xvp = jnp.zeros((LV, x_vals.shape[1]), jnp.float32).at[:nv].set(x_vals)
    xcp = jnp.zeros((LC, x_cons.shape[1]), jnp.float32).at[:nc].set(x_cons)
    hv = _enc(xvp, p["W_enc_vals"], p["b_enc_vals"], p["b_pe_vals"])
    hc = _enc(xcp, p["W_enc_cons"], p["b_enc_cons"], p["b_pe_cons"])

    agg_c2v = _sc_aggregate(nsrc=LC, nranges=5, rng=8192, seg=4736)
    agg_v2c = _sc_aggregate(nsrc=LV, nranges=1, rng=10240, seg=2368)
    g_c2v, d_c2v = _pad_edges(src_c2v, dst_c2v, nv)
    g_v2c, d_v2c = _pad_edges(src_v2c, dst_v2c, nc)

    def conv(x_src, x_dst, gsrc, dpad, agg, W1, b1, W2, b2):
        M = jnp.max(jnp.maximum(x_src, 0.0) + EPS, axis=0).reshape(1, HID)
        T = _prep(x_src, M)
        res = agg(T, gsrc, dpad)
        return _post(res, x_dst, W1, b1, W2, b2)

    hid_c, hid_v = [], []
    for _ in range(2):
        h2_v, hv_n = conv(hc, hv, g_c2v, d_c2v, agg_c2v,
                          p["W1_c2v"], p["b1_c2v"], p["W2_c2v"], p["b2_c2v"])
        h2_c, hc_n = conv(hv, hc, g_v2c, d_v2c, agg_v2c,
                          p["W1_v2c"], p["b1_v2c"], p["W2_v2c"], p["b2_v2c"])
        hid_v.append(h2_v)
        hid_c.append(h2_c)
        hv, hc = hv_n, hc_n

    ov = [_mlp(h, p["Wpv1"], p["bpv1"], p["Wpv2"], p["bpv2"])[:nv] for h in hid_v]
    oc = [_mlp(h, p["Wpc1"], p["bpc1"], p["Wpc2"], p["bpc2"])[:nc, 0] for h in hid_c]
    out_vals = jnp.stack(ov, axis=1)
    out_cons = jnp.stack(oc, axis=1)
    return (out_vals, out_cons)
